# baseline (speedup 1.0000x reference)
import jax
import jax.numpy as jnp
from jax.experimental import pallas as pl
from jax.experimental.pallas import tpu as pltpu

_CH = 4096
_GR = 512


def _body(nn_ref, q_ref, m_ref, o_ref, mbuf, rsem, ostage, wsem):
    B, Q, _ = q_ref.shape
    n_total = m_ref.shape[1]
    n_chunks = pl.cdiv(n_total, _CH)
    neg = jnp.finfo(jnp.float32).min

    gpc = _CH // _GR

    def granule_copy(bb, parity, k, g):
        base = k * _CH + g * _GR
        size = min(_GR, n_total - base)
        return pltpu.make_async_copy(
            m_ref.at[bb, base:base + size, :],
            mbuf.at[parity, k, g * _GR:g * _GR + size],
            rsem.at[parity, k, g],
        )

    def issue_reads(bb, parity):
        nnb = nn_ref[bb]
        for k in range(n_chunks):
            for g in range(gpc):
                base = k * _CH + g * _GR
                if base >= n_total:
                    break

                @pl.when(base < nnb)
                def _start():
                    granule_copy(bb, parity, k, g).start()

    issue_reads(0, 0)

    for b in range(B):
        par = b % 2
        if b + 1 < B:
            issue_reads(b + 1, (b + 1) % 2)
        if b >= 2:
            pltpu.make_async_copy(
                ostage.at[par], o_ref.at[b - 2], wsem.at[par]
            ).wait()

        nn = nn_ref[b]
        nc = pl.cdiv(nn, _CH)
        q = q_ref[b].astype(jnp.bfloat16)

        for k in range(n_chunks):
            start = k * _CH
            size = min(_CH, n_total - start)

            @pl.when(k < nc)
            def _valid():
                for g in range(gpc):
                    base = start + g * _GR
                    if base >= n_total:
                        break

                    @pl.when(base < nn)
                    def _wait():
                        granule_copy(b, par, k, g).wait()

                m = mbuf[par, k, :size].astype(jnp.bfloat16)
                logits = jax.lax.dot_general(
                    q, m, (((1,), (1,)), ((), ())),
                    preferred_element_type=jnp.float32,
                )
                n_idx = start + jax.lax.broadcasted_iota(
                    jnp.int32, logits.shape, 1
                )
                ostage[par, :, start:start + size] = jnp.where(
                    n_idx < nn, logits, neg
                )

            @pl.when(k >= nc)
            def _fill():
                ostage[par, :, start:start + size] = jnp.full(
                    (Q, size), neg, jnp.float32
                )

        pltpu.make_async_copy(
            ostage.at[par], o_ref.at[b], wsem.at[par]
        ).start()

    for b in (B - 2, B - 1):
        pltpu.make_async_copy(
            ostage.at[b % 2], o_ref.at[b], wsem.at[b % 2]
        ).wait()


def kernel(query_vector, node_nums, m_A):
    B, Q, D = query_vector.shape
    N = m_A.shape[1]
    n_chunks = pl.cdiv(N, _CH)

    grid_spec = pltpu.PrefetchScalarGridSpec(
        num_scalar_prefetch=1,
        grid=(1,),
        in_specs=[
            pl.BlockSpec((B, Q, D), lambda i, nn_ref: (0, 0, 0)),
            pl.BlockSpec(memory_space=pltpu.MemorySpace.HBM),
        ],
        out_specs=pl.BlockSpec(memory_space=pltpu.MemorySpace.HBM),
        scratch_shapes=[
            pltpu.VMEM((2, n_chunks, _CH, D), jnp.float32),
            pltpu.SemaphoreType.DMA((2, n_chunks, _CH // _GR)),
            pltpu.VMEM((2, Q, N), jnp.float32),
            pltpu.SemaphoreType.DMA((2,)),
        ],
    )
    return pl.pallas_call(
        _body,
        grid_spec=grid_spec,
        out_shape=jax.ShapeDtypeStruct((B, Q, N), jnp.float32),
    )(node_nums.astype(jnp.int32), query_vector, m_A)


# speedup vs baseline: 1.0254x; 1.0254x over previous
import jax
import jax.numpy as jnp
from jax.experimental import pallas as pl
from jax.experimental.pallas import tpu as pltpu

_CH = 1024
_GR = 512


def _body(nn_ref, q_ref, m_ref, o_ref, mbuf, rsem, ostage, wsem):
    B, Q, _ = q_ref.shape
    n_total = m_ref.shape[1]
    n_chunks = pl.cdiv(n_total, _CH)
    neg = jnp.finfo(jnp.float32).min

    gpc = _CH // _GR

    def granule_copy(bb, parity, k, g):
        base = k * _CH + g * _GR
        size = min(_GR, n_total - base)
        return pltpu.make_async_copy(
            m_ref.at[bb, base:base + size, :],
            mbuf.at[parity, k, g * _GR:g * _GR + size],
            rsem.at[parity, k, g],
        )

    def issue_reads(bb, parity):
        nnb = nn_ref[bb]
        for k in range(n_chunks):
            for g in range(gpc):
                base = k * _CH + g * _GR
                if base >= n_total:
                    break

                @pl.when(base < nnb)
                def _start():
                    granule_copy(bb, parity, k, g).start()

    issue_reads(0, 0)

    for b in range(B):
        par = b % 2
        if b + 1 < B:
            issue_reads(b + 1, (b + 1) % 2)
        if b >= 2:
            pltpu.make_async_copy(
                ostage.at[par], o_ref.at[b - 2], wsem.at[par]
            ).wait()

        nn = nn_ref[b]
        nc = pl.cdiv(nn, _CH)
        q = q_ref[b].astype(jnp.bfloat16)

        for k in range(n_chunks):
            start = k * _CH
            size = min(_CH, n_total - start)

            @pl.when(k < nc)
            def _valid():
                for g in range(gpc):
                    base = start + g * _GR
                    if base >= n_total:
                        break

                    @pl.when(base < nn)
                    def _wait():
                        granule_copy(b, par, k, g).wait()

                m = mbuf[par, k, :size].astype(jnp.bfloat16)
                logits = jax.lax.dot_general(
                    q, m, (((1,), (1,)), ((), ())),
                    preferred_element_type=jnp.float32,
                )
                n_idx = start + jax.lax.broadcasted_iota(
                    jnp.int32, logits.shape, 1
                )
                ostage[par, :, start:start + size] = jnp.where(
                    n_idx < nn, logits, neg
                )

            @pl.when(k >= nc)
            def _fill():
                ostage[par, :, start:start + size] = jnp.full(
                    (Q, size), neg, jnp.float32
                )

        pltpu.make_async_copy(
            ostage.at[par], o_ref.at[b], wsem.at[par]
        ).start()

    for b in (B - 2, B - 1):
        pltpu.make_async_copy(
            ostage.at[b % 2], o_ref.at[b], wsem.at[b % 2]
        ).wait()


def kernel(query_vector, node_nums, m_A):
    B, Q, D = query_vector.shape
    N = m_A.shape[1]
    n_chunks = pl.cdiv(N, _CH)

    grid_spec = pltpu.PrefetchScalarGridSpec(
        num_scalar_prefetch=1,
        grid=(1,),
        in_specs=[
            pl.BlockSpec((B, Q, D), lambda i, nn_ref: (0, 0, 0)),
            pl.BlockSpec(memory_space=pltpu.MemorySpace.HBM),
        ],
        out_specs=pl.BlockSpec(memory_space=pltpu.MemorySpace.HBM),
        scratch_shapes=[
            pltpu.VMEM((2, n_chunks, _CH, D), jnp.float32),
            pltpu.SemaphoreType.DMA((2, n_chunks, _CH // _GR)),
            pltpu.VMEM((2, Q, N), jnp.float32),
            pltpu.SemaphoreType.DMA((2,)),
        ],
    )
    return pl.pallas_call(
        _body,
        grid_spec=grid_spec,
        out_shape=jax.ShapeDtypeStruct((B, Q, N), jnp.float32),
    )(node_nums.astype(jnp.int32), query_vector, m_A)


# speedup vs baseline: 1.0287x; 1.0032x over previous
import jax
import jax.numpy as jnp
from jax.experimental import pallas as pl
from jax.experimental.pallas import tpu as pltpu

_CH = 2048
_GR = 1024


def _body(nn_ref, q_ref, m_ref, o_ref, mbuf, rsem, ostage, wsem):
    B, Q, _ = q_ref.shape
    n_total = m_ref.shape[1]
    n_chunks = pl.cdiv(n_total, _CH)
    neg = jnp.finfo(jnp.float32).min

    gpc = _CH // _GR

    def granule_copy(bb, parity, k, g):
        base = k * _CH + g * _GR
        size = min(_GR, n_total - base)
        return pltpu.make_async_copy(
            m_ref.at[bb, base:base + size, :],
            mbuf.at[parity, k, g * _GR:g * _GR + size],
            rsem.at[parity, k, g],
        )

    def issue_reads(bb, parity):
        nnb = nn_ref[bb]
        for k in range(n_chunks):
            for g in range(gpc):
                base = k * _CH + g * _GR
                if base >= n_total:
                    break

                @pl.when(base < nnb)
                def _start():
                    granule_copy(bb, parity, k, g).start()

    issue_reads(0, 0)

    for b in range(B):
        par = b % 2
        if b + 1 < B:
            issue_reads(b + 1, (b + 1) % 2)
        if b >= 2:
            pltpu.make_async_copy(
                ostage.at[par], o_ref.at[b - 2], wsem.at[par]
            ).wait()

        nn = nn_ref[b]
        nc = pl.cdiv(nn, _CH)
        q = q_ref[b].astype(jnp.bfloat16)

        for k in range(n_chunks):
            start = k * _CH
            size = min(_CH, n_total - start)

            @pl.when(k < nc)
            def _valid():
                for g in range(gpc):
                    base = start + g * _GR
                    if base >= n_total:
                        break

                    @pl.when(base < nn)
                    def _wait():
                        granule_copy(b, par, k, g).wait()

                m = mbuf[par, k, :size].astype(jnp.bfloat16)
                logits = jax.lax.dot_general(
                    q, m, (((1,), (1,)), ((), ())),
                    preferred_element_type=jnp.float32,
                )
                n_idx = start + jax.lax.broadcasted_iota(
                    jnp.int32, logits.shape, 1
                )
                ostage[par, :, start:start + size] = jnp.where(
                    n_idx < nn, logits, neg
                )

            @pl.when(k >= nc)
            def _fill():
                ostage[par, :, start:start + size] = jnp.full(
                    (Q, size), neg, jnp.float32
                )

        pltpu.make_async_copy(
            ostage.at[par], o_ref.at[b], wsem.at[par]
        ).start()

    for b in (B - 2, B - 1):
        pltpu.make_async_copy(
            ostage.at[b % 2], o_ref.at[b], wsem.at[b % 2]
        ).wait()


def kernel(query_vector, node_nums, m_A):
    B, Q, D = query_vector.shape
    N = m_A.shape[1]
    n_chunks = pl.cdiv(N, _CH)

    grid_spec = pltpu.PrefetchScalarGridSpec(
        num_scalar_prefetch=1,
        grid=(1,),
        in_specs=[
            pl.BlockSpec((B, Q, D), lambda i, nn_ref: (0, 0, 0)),
            pl.BlockSpec(memory_space=pltpu.MemorySpace.HBM),
        ],
        out_specs=pl.BlockSpec(memory_space=pltpu.MemorySpace.HBM),
        scratch_shapes=[
            pltpu.VMEM((2, n_chunks, _CH, D), jnp.float32),
            pltpu.SemaphoreType.DMA((2, n_chunks, _CH // _GR)),
            pltpu.VMEM((2, Q, N), jnp.float32),
            pltpu.SemaphoreType.DMA((2,)),
        ],
    )
    return pl.pallas_call(
        _body,
        grid_spec=grid_spec,
        out_shape=jax.ShapeDtypeStruct((B, Q, N), jnp.float32),
    )(node_nums.astype(jnp.int32), query_vector, m_A)
